# baseline (speedup 1.0000x reference)
import functools

import jax
import jax.numpy as jnp
from jax import lax
from jax.experimental import pallas as pl
from jax.experimental.pallas import tpu as pltpu
from jax.experimental.pallas import tpu_sc as plsc

_NC = 2
_NS = 16
_NW = _NC * _NS
_CH = 128
_WAVE = 16


def kernel(user_ids, table):
    B = user_ids.shape[0]
    V, D = table.shape
    b_per_w = B // _NW
    elems_w = D * b_per_w
    n_waves = elems_w // (_WAVE * _CH)

    zero = (user_ids[0] & 0).astype(jnp.float32)
    table_flat = table.T.reshape(-1) + zero

    uids = user_ids.astype(jnp.int32).reshape(_NW, 1, b_per_w)
    feat = (jnp.arange(D, dtype=jnp.int32) * V).reshape(1, D, 1)
    abs_ids = (uids + feat).reshape(_NW, elems_w)

    mesh = plsc.VectorSubcoreMesh(core_axis_name="c", subcore_axis_name="s")

    @functools.partial(
        pl.kernel,
        out_type=jax.ShapeDtypeStruct((D * B,), jnp.float32),
        mesh=mesh,
        scratch_types=[
            pltpu.VMEM((elems_w,), jnp.int32),
            pltpu.VMEM((elems_w,), jnp.float32),
            pltpu.SemaphoreType.DMA,
            pltpu.SemaphoreType.DMA,
        ],
    )
    def gather_kernel(ids_hbm, table_hbm, out_hbm, idx_v, acc_v, gsem, wsem):
        wid = lax.axis_index("s") * _NC + lax.axis_index("c")
        pltpu.sync_copy(ids_hbm.at[wid], idx_v)

        def wave(w, _):
            base = pl.multiple_of(w * (_WAVE * _CH), _WAVE * _CH)
            copies = []
            for k in range(_WAVE):
                off = pl.multiple_of(base + k * _CH, _CH)
                copies.append(
                    pltpu.async_copy(
                        table_hbm.at[idx_v.at[pl.ds(off, _CH)]],
                        acc_v.at[pl.ds(off, _CH)],
                        gsem,
                    )
                )
            for c in copies:
                c.wait()
            return 0

        lax.fori_loop(0, n_waves, wave, 0)

        def writeback(i, _):
            copies = []
            for k in range(4):
                f = i * 4 + k
                src = pl.multiple_of(f * b_per_w, b_per_w)
                dst = pl.multiple_of(f * B + wid * b_per_w, b_per_w)
                copies.append(
                    pltpu.async_copy(
                        acc_v.at[pl.ds(src, b_per_w)],
                        out_hbm.at[pl.ds(dst, b_per_w)],
                        wsem,
                    )
                )
            for c in copies:
                c.wait()
            return 0

        lax.fori_loop(0, D // 4, writeback, 0)

    out_flat = gather_kernel(abs_ids, table_flat)
    return out_flat.reshape(D, B).T
